# baseline (speedup 1.0000x reference)
import functools

import jax
import jax.numpy as jnp
from jax import lax
from jax.experimental import pallas as pl
from jax.experimental.pallas import tpu as pltpu
from jax.experimental.pallas import tpu_sc as plsc

S = 2048
D_MODEL = 1024
NUM_EXPERTS = 8
TOP_K = 2
D_FF = 4096
FFC = 2048
NF = D_FF // FFC
BM = 256
NP = S * TOP_K
NB_REAL = NP // BM + NUM_EXPERTS
NB_TOTAL = NB_REAL + 1
PT = NB_TOTAL * BM

SC_NC = 2
SC_NS = 16
SC_NW = SC_NC * SC_NS


def _router_body(x_ref, wr_ref, br_ref, idx_ref, wn_ref, aux_ref):
    x = x_ref[...]
    logits = lax.dot_general(
        x, wr_ref[...], (((1,), (0,)), ((), ())),
        preferred_element_type=jnp.float32) + br_ref[...]
    m = jnp.max(logits, axis=-1, keepdims=True)
    ex = jnp.exp(logits - m)
    probs = ex / jnp.sum(ex, axis=-1, keepdims=True)

    iota = lax.broadcasted_iota(jnp.int32, probs.shape, 1)
    m1 = jnp.max(probs, axis=-1, keepdims=True)
    idx1 = jnp.min(jnp.where(probs == m1, iota, NUM_EXPERTS), axis=-1,
                   keepdims=True)
    p2 = jnp.where(iota == idx1, -jnp.inf, probs)
    m2 = jnp.max(p2, axis=-1, keepdims=True)
    idx2 = jnp.min(jnp.where(p2 == m2, iota, NUM_EXPERTS), axis=-1,
                   keepdims=True)
    denom = jnp.maximum(m1 + m2, 1e-9)

    idx_ref[...] = jnp.concatenate([idx1, idx2], axis=1)
    wn_ref[...] = jnp.concatenate([m1 / denom, m2 / denom], axis=1)

    f_cnt = jnp.sum(jnp.where(iota == idx1, 1.0, 0.0), axis=0, keepdims=True)
    p_sum = jnp.sum(probs, axis=0, keepdims=True)
    aux = (NUM_EXPERTS / (S * S)) * jnp.sum(f_cnt * p_sum, keepdims=True)
    aux_ref[...] = aux.reshape(1, 1)


def _run_router(x2, Wr, br):
    return pl.pallas_call(
        _router_body,
        out_shape=[
            jax.ShapeDtypeStruct((S, TOP_K), jnp.int32),
            jax.ShapeDtypeStruct((S, TOP_K), jnp.float32),
            jax.ShapeDtypeStruct((1, 1), jnp.float32),
        ],
    )(x2, Wr, br.reshape(1, NUM_EXPERTS))


def _schedule(idxs, wns):
    pair_e = jnp.concatenate([idxs[:, 0], idxs[:, 1]])
    pair_t = jnp.tile(jnp.arange(S, dtype=jnp.int32), 2)
    pair_w = jnp.concatenate([wns[:, 0], wns[:, 1]])

    onehot = (pair_e[:, None] == jnp.arange(NUM_EXPERTS)[None, :])
    counts = jnp.sum(onehot, axis=0, dtype=jnp.int32)
    rank = jnp.take_along_axis(
        jnp.cumsum(onehot, axis=0, dtype=jnp.int32) - 1,
        pair_e[:, None].astype(jnp.int32), axis=1)[:, 0]

    nblk = (counts + BM - 1) // BM
    bstart = jnp.concatenate(
        [jnp.zeros((1,), jnp.int32), jnp.cumsum(nblk)[:-1]]).astype(jnp.int32)
    pos = bstart[pair_e] * BM + rank

    row_ids = jnp.zeros((PT,), jnp.int32).at[pos].set(pair_t)
    w_pad = jnp.zeros((PT,), jnp.float32).at[pos].set(pair_w)
    pos1 = pos[:S]
    pos2 = pos[S:]

    cum = jnp.cumsum(nblk)
    bid = jnp.arange(NB_REAL, dtype=jnp.int32)
    eb_raw = jnp.sum(cum[None, :] <= bid[:, None], axis=1,
                     dtype=jnp.int32)
    total = cum[-1]
    valid = (bid < total).astype(jnp.int32)
    eb_last = jnp.take(eb_raw, jnp.maximum(total - 1, 0))
    be = jnp.where(valid == 1, jnp.minimum(eb_raw, NUM_EXPERTS - 1), eb_last)
    return row_ids, w_pad, be.astype(jnp.int32), valid, pos1, pos2


_G_PER_W = PT // SC_NW
_G_CH = 40
_G_NCH = _G_PER_W // _G_CH


def _gmm_body(be_ref, valid_ref, ids_ref, x_ref, w1_ref, b1_ref, w2_ref,
              b2_ref, wp_ref, y_ref, xbuf):
    b = pl.program_id(0)
    f = pl.program_id(1)

    @pl.when(valid_ref[b] == 1)
    def _():
        @pl.when(f == 0)
        def _():
            ids = ids_ref[...]
            cols = lax.broadcasted_iota(jnp.int32, (BM, S), 1)
            perm = jnp.where(cols == ids, 1.0, 0.0)
            xbuf[...] = lax.dot_general(
                perm, x_ref[...], (((1,), (0,)), ((), ())),
                preferred_element_type=jnp.float32)

        xv = xbuf[...]
        wcol = wp_ref[...]
        h = lax.dot_general(
            xv, w1_ref[0], (((1,), (0,)), ((), ())),
            preferred_element_type=jnp.float32) + b1_ref[0]
        h = 0.5 * h * (1.0 + lax.erf(h * (2.0 ** -0.5)))
        h = h * wcol
        contrib = lax.dot_general(
            h, w2_ref[0], (((1,), (0,)), ((), ())),
            preferred_element_type=jnp.float32)

        @pl.when(f == 0)
        def _():
            y_ref[...] = contrib + b2_ref[0] * wcol

        @pl.when(f > 0)
        def _():
            y_ref[...] += contrib


def _grouped_matmul(be, valid, row_ids, x2, W1, b1, W2, b2, w_pad):
    TRASH = NB_TOTAL - 1

    def y_map(b, f, be, valid):
        return (jnp.where(valid[b] == 1, b, TRASH), 0)

    grid_spec = pltpu.PrefetchScalarGridSpec(
        num_scalar_prefetch=2,
        grid=(NB_REAL, NF),
        in_specs=[
            pl.BlockSpec((BM, 1), lambda b, f, be, v: (b, 0)),
            pl.BlockSpec((S, D_MODEL), lambda b, f, be, v: (0, 0)),
            pl.BlockSpec((1, D_MODEL, FFC), lambda b, f, be, v: (be[b], 0, f)),
            pl.BlockSpec((1, 1, FFC), lambda b, f, be, v: (be[b], 0, f)),
            pl.BlockSpec((1, FFC, D_MODEL), lambda b, f, be, v: (be[b], f, 0)),
            pl.BlockSpec((1, 1, D_MODEL), lambda b, f, be, v: (be[b], 0, 0)),
            pl.BlockSpec((BM, 1), lambda b, f, be, v: (b, 0)),
        ],
        out_specs=pl.BlockSpec((BM, D_MODEL), y_map),
        scratch_shapes=[
            pltpu.VMEM((BM, D_MODEL), jnp.float32),
        ],
    )
    return pl.pallas_call(
        _gmm_body,
        grid_spec=grid_spec,
        out_shape=jax.ShapeDtypeStruct((PT, D_MODEL), jnp.float32),
    )(be, valid, row_ids.reshape(PT, 1), x2,
      W1, b1.reshape(NUM_EXPERTS, 1, D_FF),
      W2, b2.reshape(NUM_EXPERTS, 1, D_MODEL),
      w_pad.reshape(PT, 1))


_C_PER_W = S // SC_NW
_C_CH = 32
_C_NCH = _C_PER_W // _C_CH
_ROW_VR = D_MODEL // 16


def _sc_combine_body(p1_hbm, p2_hbm, y_hbm, out_hbm, p1_v, p2_v, r1_v, r2_v,
                     sem, sem2):
    wid = lax.axis_index("s") * SC_NC + lax.axis_index("c")
    base = wid * _C_PER_W
    for c in range(_C_NCH):
        off = base + c * _C_CH
        pltpu.sync_copy(p1_hbm.at[pl.ds(off, _C_CH)], p1_v)
        pltpu.sync_copy(p2_hbm.at[pl.ds(off, _C_CH)], p2_v)
        d1 = pltpu.async_copy(y_hbm.at[p1_v], r1_v, sem)
        d2 = pltpu.async_copy(y_hbm.at[p2_v], r2_v, sem2)
        d1.wait()
        d2.wait()

        def row_add(j, _):
            for k in range(_ROW_VR):
                sl = pl.ds(k * 16, 16)
                r1_v[j, sl] = r1_v[j, sl] + r2_v[j, sl]
            return 0

        lax.fori_loop(0, _C_CH, row_add, 0)
        pltpu.sync_copy(r1_v, out_hbm.at[pl.ds(off, _C_CH)])


def _sc_combine(pos1, pos2, y):
    mesh = plsc.VectorSubcoreMesh(core_axis_name="c", subcore_axis_name="s")
    return pl.kernel(
        _sc_combine_body,
        mesh=mesh,
        out_type=jax.ShapeDtypeStruct((S, D_MODEL), jnp.float32),
        scratch_types=[
            pltpu.VMEM((_C_CH,), jnp.int32),
            pltpu.VMEM((_C_CH,), jnp.int32),
            pltpu.VMEM((_C_CH, D_MODEL), jnp.float32),
            pltpu.VMEM((_C_CH, D_MODEL), jnp.float32),
            pltpu.SemaphoreType.DMA,
            pltpu.SemaphoreType.DMA,
        ],
    )(pos1, pos2, y)


@jax.jit
def kernel(x, Wr, br, W1, b1, W2, b2):
    B = x.shape[0]
    x2 = x.reshape(S, D_MODEL)

    idxs, wns, aux = _run_router(x2, Wr, br)
    row_ids, w_pad, be, valid, pos1, pos2 = _schedule(idxs, wns)
    y = _grouped_matmul(be, valid, row_ids, x2, W1, b1, W2, b2, w_pad)
    out2 = _sc_combine(pos1, pos2, y)
    return out2.reshape(B, S, D_MODEL), aux[0, 0]


# speedup vs baseline: 1.1726x; 1.1726x over previous
import functools

import jax
import jax.numpy as jnp
from jax import lax
from jax.experimental import pallas as pl
from jax.experimental.pallas import tpu as pltpu
from jax.experimental.pallas import tpu_sc as plsc

S = 2048
D_MODEL = 1024
NUM_EXPERTS = 8
TOP_K = 2
D_FF = 4096
FFC = 2048
NF = D_FF // FFC
BM = 512
NP = S * TOP_K
NB_REAL = NP // BM + NUM_EXPERTS
NB_TOTAL = NB_REAL + 1
PT = NB_TOTAL * BM

SC_NC = 2
SC_NS = 16
SC_NW = SC_NC * SC_NS


def _router_body(x_ref, wr_ref, br_ref, idx_ref, wn_ref, aux_ref):
    x = x_ref[...]
    logits = lax.dot_general(
        x, wr_ref[...], (((1,), (0,)), ((), ())),
        preferred_element_type=jnp.float32) + br_ref[...]
    m = jnp.max(logits, axis=-1, keepdims=True)
    ex = jnp.exp(logits - m)
    probs = ex / jnp.sum(ex, axis=-1, keepdims=True)

    iota = lax.broadcasted_iota(jnp.int32, probs.shape, 1)
    m1 = jnp.max(probs, axis=-1, keepdims=True)
    idx1 = jnp.min(jnp.where(probs == m1, iota, NUM_EXPERTS), axis=-1,
                   keepdims=True)
    p2 = jnp.where(iota == idx1, -jnp.inf, probs)
    m2 = jnp.max(p2, axis=-1, keepdims=True)
    idx2 = jnp.min(jnp.where(p2 == m2, iota, NUM_EXPERTS), axis=-1,
                   keepdims=True)
    denom = jnp.maximum(m1 + m2, 1e-9)

    idx_ref[...] = jnp.concatenate([idx1, idx2], axis=1)
    wn_ref[...] = jnp.concatenate([m1 / denom, m2 / denom], axis=1)

    f_cnt = jnp.sum(jnp.where(iota == idx1, 1.0, 0.0), axis=0, keepdims=True)
    p_sum = jnp.sum(probs, axis=0, keepdims=True)
    aux = (NUM_EXPERTS / (S * S)) * jnp.sum(f_cnt * p_sum, keepdims=True)
    aux_ref[...] = aux.reshape(1, 1)


def _run_router(x2, Wr, br):
    return pl.pallas_call(
        _router_body,
        out_shape=[
            jax.ShapeDtypeStruct((S, TOP_K), jnp.int32),
            jax.ShapeDtypeStruct((S, TOP_K), jnp.float32),
            jax.ShapeDtypeStruct((1, 1), jnp.float32),
        ],
    )(x2, Wr, br.reshape(1, NUM_EXPERTS))


def _schedule(idxs, wns):
    pair_e = jnp.concatenate([idxs[:, 0], idxs[:, 1]])
    pair_t = jnp.tile(jnp.arange(S, dtype=jnp.int32), 2)
    pair_w = jnp.concatenate([wns[:, 0], wns[:, 1]])

    onehot = (pair_e[:, None] == jnp.arange(NUM_EXPERTS)[None, :])
    counts = jnp.sum(onehot, axis=0, dtype=jnp.int32)
    rank = jnp.take_along_axis(
        jnp.cumsum(onehot, axis=0, dtype=jnp.int32) - 1,
        pair_e[:, None].astype(jnp.int32), axis=1)[:, 0]

    nblk = (counts + BM - 1) // BM
    bstart = jnp.concatenate(
        [jnp.zeros((1,), jnp.int32), jnp.cumsum(nblk)[:-1]]).astype(jnp.int32)
    pos = bstart[pair_e] * BM + rank

    row_ids = jnp.zeros((PT,), jnp.int32).at[pos].set(pair_t)
    w_pad = jnp.zeros((PT,), jnp.float32).at[pos].set(pair_w)
    pos1 = pos[:S]
    pos2 = pos[S:]

    cum = jnp.cumsum(nblk)
    bid = jnp.arange(NB_REAL, dtype=jnp.int32)
    eb_raw = jnp.sum(cum[None, :] <= bid[:, None], axis=1,
                     dtype=jnp.int32)
    total = cum[-1]
    valid = (bid < total).astype(jnp.int32)
    eb_last = jnp.take(eb_raw, jnp.maximum(total - 1, 0))
    be = jnp.where(valid == 1, jnp.minimum(eb_raw, NUM_EXPERTS - 1), eb_last)
    return row_ids, w_pad, be.astype(jnp.int32), valid, pos1, pos2


_G_PER_W = PT // SC_NW
_G_CH = 40
_G_NCH = _G_PER_W // _G_CH


def _gmm_body(be_ref, valid_ref, ids_ref, x_ref, w1_ref, b1_ref, w2_ref,
              b2_ref, wp_ref, y_ref, xbuf):
    b = pl.program_id(0)
    f = pl.program_id(1)

    @pl.when(valid_ref[b] == 1)
    def _():
        @pl.when(f == 0)
        def _():
            ids = ids_ref[...]
            cols = lax.broadcasted_iota(jnp.int32, (BM, S), 1)
            perm = jnp.where(cols == ids, 1.0, 0.0)
            xbuf[...] = lax.dot_general(
                perm, x_ref[...], (((1,), (0,)), ((), ())),
                preferred_element_type=jnp.float32)

        xv = xbuf[...]
        wcol = wp_ref[...]
        h = lax.dot_general(
            xv, w1_ref[0], (((1,), (0,)), ((), ())),
            preferred_element_type=jnp.float32) + b1_ref[0]
        h = 0.5 * h * (1.0 + lax.erf(h * (2.0 ** -0.5)))
        h = h * wcol
        contrib = lax.dot_general(
            h, w2_ref[0], (((1,), (0,)), ((), ())),
            preferred_element_type=jnp.float32)

        @pl.when(f == 0)
        def _():
            y_ref[...] = contrib + b2_ref[0] * wcol

        @pl.when(f > 0)
        def _():
            y_ref[...] += contrib


def _grouped_matmul(be, valid, row_ids, x2, W1, b1, W2, b2, w_pad):
    TRASH = NB_TOTAL - 1

    def y_map(b, f, be, valid):
        return (jnp.where(valid[b] == 1, b, TRASH), 0)

    grid_spec = pltpu.PrefetchScalarGridSpec(
        num_scalar_prefetch=2,
        grid=(NB_REAL, NF),
        in_specs=[
            pl.BlockSpec((BM, 1), lambda b, f, be, v: (b, 0)),
            pl.BlockSpec((S, D_MODEL), lambda b, f, be, v: (0, 0)),
            pl.BlockSpec((1, D_MODEL, FFC), lambda b, f, be, v: (be[b], 0, f)),
            pl.BlockSpec((1, 1, FFC), lambda b, f, be, v: (be[b], 0, f)),
            pl.BlockSpec((1, FFC, D_MODEL), lambda b, f, be, v: (be[b], f, 0)),
            pl.BlockSpec((1, 1, D_MODEL), lambda b, f, be, v: (be[b], 0, 0)),
            pl.BlockSpec((BM, 1), lambda b, f, be, v: (b, 0)),
        ],
        out_specs=pl.BlockSpec((BM, D_MODEL), y_map),
        scratch_shapes=[
            pltpu.VMEM((BM, D_MODEL), jnp.float32),
        ],
    )
    return pl.pallas_call(
        _gmm_body,
        grid_spec=grid_spec,
        out_shape=jax.ShapeDtypeStruct((PT, D_MODEL), jnp.float32),
    )(be, valid, row_ids.reshape(PT, 1), x2,
      W1, b1.reshape(NUM_EXPERTS, 1, D_FF),
      W2, b2.reshape(NUM_EXPERTS, 1, D_MODEL),
      w_pad.reshape(PT, 1))


_C_PER_W = S // SC_NW
_C_CH = 32
_C_NCH = _C_PER_W // _C_CH
_ROW_VR = D_MODEL // 16


def _sc_combine_body(p1_hbm, p2_hbm, y_hbm, out_hbm, p1_v, p2_v, r1_v, r2_v,
                     sem, sem2):
    wid = lax.axis_index("s") * SC_NC + lax.axis_index("c")
    base = wid * _C_PER_W
    for c in range(_C_NCH):
        off = base + c * _C_CH
        pltpu.sync_copy(p1_hbm.at[pl.ds(off, _C_CH)], p1_v)
        pltpu.sync_copy(p2_hbm.at[pl.ds(off, _C_CH)], p2_v)
        d1 = pltpu.async_copy(y_hbm.at[p1_v], r1_v, sem)
        d2 = pltpu.async_copy(y_hbm.at[p2_v], r2_v, sem2)
        d1.wait()
        d2.wait()

        def row_add(j, _):
            for k in range(_ROW_VR):
                sl = pl.ds(k * 16, 16)
                r1_v[j, sl] = r1_v[j, sl] + r2_v[j, sl]
            return 0

        lax.fori_loop(0, _C_CH, row_add, 0)
        pltpu.sync_copy(r1_v, out_hbm.at[pl.ds(off, _C_CH)])


def _sc_combine(pos1, pos2, y):
    mesh = plsc.VectorSubcoreMesh(core_axis_name="c", subcore_axis_name="s")
    return pl.kernel(
        _sc_combine_body,
        mesh=mesh,
        out_type=jax.ShapeDtypeStruct((S, D_MODEL), jnp.float32),
        scratch_types=[
            pltpu.VMEM((_C_CH,), jnp.int32),
            pltpu.VMEM((_C_CH,), jnp.int32),
            pltpu.VMEM((_C_CH, D_MODEL), jnp.float32),
            pltpu.VMEM((_C_CH, D_MODEL), jnp.float32),
            pltpu.SemaphoreType.DMA,
            pltpu.SemaphoreType.DMA,
        ],
    )(pos1, pos2, y)


@jax.jit
def kernel(x, Wr, br, W1, b1, W2, b2):
    B = x.shape[0]
    x2 = x.reshape(S, D_MODEL)

    idxs, wns, aux = _run_router(x2, Wr, br)
    row_ids, w_pad, be, valid, pos1, pos2 = _schedule(idxs, wns)
    y = _grouped_matmul(be, valid, row_ids, x2, W1, b1, W2, b2, w_pad)
    out2 = _sc_combine(pos1, pos2, y)
    return out2.reshape(B, S, D_MODEL), aux[0, 0]


# speedup vs baseline: 1.3463x; 1.1481x over previous
import functools

import jax
import jax.numpy as jnp
from jax import lax
from jax.experimental import pallas as pl
from jax.experimental.pallas import tpu as pltpu
from jax.experimental.pallas import tpu_sc as plsc

S = 2048
D_MODEL = 1024
NUM_EXPERTS = 8
TOP_K = 2
D_FF = 4096
FFC = 2048
NF = D_FF // FFC
BM = 512
NP = S * TOP_K
NB_REAL = NP // BM + NUM_EXPERTS
NB_TOTAL = NB_REAL + 1
PT = NB_TOTAL * BM

SC_NC = 2
SC_NS = 16
SC_NW = SC_NC * SC_NS


def _router_body(x_ref, wr_ref, br_ref, idx_ref, wn_ref, aux_ref):
    x = x_ref[...]
    logits = lax.dot_general(
        x, wr_ref[...], (((1,), (0,)), ((), ())),
        preferred_element_type=jnp.float32) + br_ref[...]
    m = jnp.max(logits, axis=-1, keepdims=True)
    ex = jnp.exp(logits - m)
    probs = ex / jnp.sum(ex, axis=-1, keepdims=True)

    iota = lax.broadcasted_iota(jnp.int32, probs.shape, 1)
    m1 = jnp.max(probs, axis=-1, keepdims=True)
    idx1 = jnp.min(jnp.where(probs == m1, iota, NUM_EXPERTS), axis=-1,
                   keepdims=True)
    p2 = jnp.where(iota == idx1, -jnp.inf, probs)
    m2 = jnp.max(p2, axis=-1, keepdims=True)
    idx2 = jnp.min(jnp.where(p2 == m2, iota, NUM_EXPERTS), axis=-1,
                   keepdims=True)
    denom = jnp.maximum(m1 + m2, 1e-9)

    idx_ref[...] = jnp.concatenate([idx1, idx2], axis=1)
    wn_ref[...] = jnp.concatenate([m1 / denom, m2 / denom], axis=1)

    f_cnt = jnp.sum(jnp.where(iota == idx1, 1.0, 0.0), axis=0, keepdims=True)
    p_sum = jnp.sum(probs, axis=0, keepdims=True)
    aux = (NUM_EXPERTS / (S * S)) * jnp.sum(f_cnt * p_sum, keepdims=True)
    aux_ref[...] = aux.reshape(1, 1)


def _run_router(x2, Wr, br):
    return pl.pallas_call(
        _router_body,
        out_shape=[
            jax.ShapeDtypeStruct((S, TOP_K), jnp.int32),
            jax.ShapeDtypeStruct((S, TOP_K), jnp.float32),
            jax.ShapeDtypeStruct((1, 1), jnp.float32),
        ],
    )(x2, Wr, br.reshape(1, NUM_EXPERTS))


def _schedule(idxs, wns):
    pair_e = jnp.concatenate([idxs[:, 0], idxs[:, 1]])
    pair_t = jnp.tile(jnp.arange(S, dtype=jnp.int32), 2)
    pair_w = jnp.concatenate([wns[:, 0], wns[:, 1]])

    onehot = (pair_e[:, None] == jnp.arange(NUM_EXPERTS)[None, :])
    counts = jnp.sum(onehot, axis=0, dtype=jnp.int32)
    rank = jnp.take_along_axis(
        jnp.cumsum(onehot, axis=0, dtype=jnp.int32) - 1,
        pair_e[:, None].astype(jnp.int32), axis=1)[:, 0]

    nblk = (counts + BM - 1) // BM
    bstart = jnp.concatenate(
        [jnp.zeros((1,), jnp.int32), jnp.cumsum(nblk)[:-1]]).astype(jnp.int32)
    pos = bstart[pair_e] * BM + rank

    del pair_t, pair_w
    pos1 = pos[:S]
    pos2 = pos[S:]

    cum = jnp.cumsum(nblk)
    bid = jnp.arange(NB_REAL, dtype=jnp.int32)
    eb_raw = jnp.sum(cum[None, :] <= bid[:, None], axis=1,
                     dtype=jnp.int32)
    total = cum[-1]
    valid = (bid < total).astype(jnp.int32)
    eb_last = jnp.take(eb_raw, jnp.maximum(total - 1, 0))
    be = jnp.where(valid == 1, jnp.minimum(eb_raw, NUM_EXPERTS - 1), eb_last)
    return be.astype(jnp.int32), valid, pos1, pos2


_G_PER_W = PT // SC_NW
_G_CH = 40
_G_NCH = _G_PER_W // _G_CH


def _gmm_body(be_ref, valid_ref, p1_ref, p2_ref, q1_ref, q2_ref, x_ref,
              w1_ref, b1_ref, w2_ref, b2_ref, y_ref, xbuf, wbuf):
    b = pl.program_id(0)
    f = pl.program_id(1)

    @pl.when(valid_ref[b] == 1)
    def _():
        @pl.when(f == 0)
        def _():
            slots = (lax.broadcasted_iota(jnp.int32, (BM, S), 0)
                     + b * BM)
            m1 = slots == p1_ref[...]
            m2 = slots == p2_ref[...]
            perm = jnp.where(m1 | m2, 1.0, 0.0)
            xbuf[...] = lax.dot_general(
                perm, x_ref[...], (((1,), (0,)), ((), ())),
                preferred_element_type=jnp.float32)
            permw = (jnp.where(m1, q1_ref[...], 0.0)
                     + jnp.where(m2, q2_ref[...], 0.0))
            wbuf[...] = jnp.sum(permw, axis=1, keepdims=True)

        xv = xbuf[...]
        wcol = wbuf[...]
        h = lax.dot_general(
            xv, w1_ref[0], (((1,), (0,)), ((), ())),
            preferred_element_type=jnp.float32) + b1_ref[0]
        h = 0.5 * h * (1.0 + lax.erf(h * (2.0 ** -0.5)))
        h = h * wcol
        contrib = lax.dot_general(
            h, w2_ref[0], (((1,), (0,)), ((), ())),
            preferred_element_type=jnp.float32)

        @pl.when(f == 0)
        def _():
            y_ref[...] = contrib + b2_ref[0] * wcol

        @pl.when(f > 0)
        def _():
            y_ref[...] += contrib


def _grouped_matmul(be, valid, pos1, pos2, wns, x2, W1, b1, W2, b2):
    TRASH = NB_TOTAL - 1

    def y_map(b, f, be, valid):
        return (jnp.where(valid[b] == 1, b, TRASH), 0)

    grid_spec = pltpu.PrefetchScalarGridSpec(
        num_scalar_prefetch=2,
        grid=(NB_REAL, NF),
        in_specs=[
            pl.BlockSpec((1, S), lambda b, f, be, v: (0, 0)),
            pl.BlockSpec((1, S), lambda b, f, be, v: (0, 0)),
            pl.BlockSpec((1, S), lambda b, f, be, v: (0, 0)),
            pl.BlockSpec((1, S), lambda b, f, be, v: (0, 0)),
            pl.BlockSpec((S, D_MODEL), lambda b, f, be, v: (0, 0)),
            pl.BlockSpec((1, D_MODEL, FFC), lambda b, f, be, v: (be[b], 0, f)),
            pl.BlockSpec((1, 1, FFC), lambda b, f, be, v: (be[b], 0, f)),
            pl.BlockSpec((1, FFC, D_MODEL), lambda b, f, be, v: (be[b], f, 0)),
            pl.BlockSpec((1, 1, D_MODEL), lambda b, f, be, v: (be[b], 0, 0)),
        ],
        out_specs=pl.BlockSpec((BM, D_MODEL), y_map),
        scratch_shapes=[
            pltpu.VMEM((BM, D_MODEL), jnp.float32),
            pltpu.VMEM((BM, 1), jnp.float32),
        ],
    )
    return pl.pallas_call(
        _gmm_body,
        grid_spec=grid_spec,
        out_shape=jax.ShapeDtypeStruct((PT, D_MODEL), jnp.float32),
    )(be, valid, pos1.reshape(1, S), pos2.reshape(1, S),
      wns[:, 0].reshape(1, S), wns[:, 1].reshape(1, S), x2,
      W1, b1.reshape(NUM_EXPERTS, 1, D_FF),
      W2, b2.reshape(NUM_EXPERTS, 1, D_MODEL))


_C_PER_W = S // SC_NW
_C_CH = 32
_C_NCH = _C_PER_W // _C_CH
_ROW_VR = D_MODEL // 16


def _sc_combine_body(p1_hbm, p2_hbm, y_hbm, out_hbm, p1_v, p2_v, r1_v, r2_v,
                     sem, sem2):
    wid = lax.axis_index("s") * SC_NC + lax.axis_index("c")
    base = wid * _C_PER_W
    for c in range(_C_NCH):
        off = base + c * _C_CH
        pltpu.sync_copy(p1_hbm.at[pl.ds(off, _C_CH)], p1_v)
        pltpu.sync_copy(p2_hbm.at[pl.ds(off, _C_CH)], p2_v)
        d1 = pltpu.async_copy(y_hbm.at[p1_v], r1_v, sem)
        d2 = pltpu.async_copy(y_hbm.at[p2_v], r2_v, sem2)
        d1.wait()
        d2.wait()

        def row_add(j, _):
            for k in range(_ROW_VR):
                sl = pl.ds(k * 16, 16)
                r1_v[j, sl] = r1_v[j, sl] + r2_v[j, sl]
            return 0

        lax.fori_loop(0, _C_CH, row_add, 0)
        pltpu.sync_copy(r1_v, out_hbm.at[pl.ds(off, _C_CH)])


def _sc_combine(pos1, pos2, y):
    mesh = plsc.VectorSubcoreMesh(core_axis_name="c", subcore_axis_name="s")
    return pl.kernel(
        _sc_combine_body,
        mesh=mesh,
        out_type=jax.ShapeDtypeStruct((S, D_MODEL), jnp.float32),
        scratch_types=[
            pltpu.VMEM((_C_CH,), jnp.int32),
            pltpu.VMEM((_C_CH,), jnp.int32),
            pltpu.VMEM((_C_CH, D_MODEL), jnp.float32),
            pltpu.VMEM((_C_CH, D_MODEL), jnp.float32),
            pltpu.SemaphoreType.DMA,
            pltpu.SemaphoreType.DMA,
        ],
    )(pos1, pos2, y)


@jax.jit
def kernel(x, Wr, br, W1, b1, W2, b2):
    B = x.shape[0]
    x2 = x.reshape(S, D_MODEL)

    idxs, wns, aux = _run_router(x2, Wr, br)
    be, valid, pos1, pos2 = _schedule(idxs, wns)
    y = _grouped_matmul(be, valid, pos1, pos2, wns, x2, W1, b1, W2, b2)
    out2 = _sc_combine(pos1, pos2, y)
    return out2.reshape(B, S, D_MODEL), aux[0, 0]
